# baseline (speedup 1.0000x reference)
import functools

import jax
import jax.numpy as jnp
from jax import lax
from jax.experimental import pallas as pl
from jax.experimental.pallas import tpu as pltpu
from jax.experimental.pallas import tpu_sc as plsc

_B = 8
_N = 8192
_NC = 512
_EPS = 1e-5

_SC_CORES = 2
_SC_SUBCORES = 16
_NW = _SC_CORES * _SC_SUBCORES
_ROWS_PER_W = _B * _NC // _NW


def _fps_body(xyz_ref, far0_ref, idx_ref, nx0_ref, nx1_ref, nx2_ref, dist_ref):
    x0 = xyz_ref[:, 0, :]
    x1 = xyz_ref[:, 1, :]
    x2 = xyz_ref[:, 2, :]
    lane = jax.lax.broadcasted_iota(jnp.int32, (_B, _N), 1)
    col = jax.lax.broadcasted_iota(jnp.int32, (_B, _NC), 1)
    dist_ref[...] = jnp.full((_B, _N), 1e10, jnp.float32)
    idx_ref[...] = jnp.zeros((_B, _NC), jnp.int32)
    nx0_ref[...] = jnp.zeros((_B, _NC), jnp.float32)
    nx1_ref[...] = jnp.zeros((_B, _NC), jnp.float32)
    nx2_ref[...] = jnp.zeros((_B, _NC), jnp.float32)
    x24 = jnp.concatenate([x0, x1, x2], axis=0)

    def body(i, far):
        sel = lane == far
        sel24 = jnp.concatenate([sel, sel, sel], axis=0)
        g = jnp.sum(jnp.where(sel24, x24, 0.0), axis=1, keepdims=True)
        c0 = g[0:_B]
        c1 = g[_B:2 * _B]
        c2 = g[2 * _B:3 * _B]
        d0 = x0 - c0
        d1 = x1 - c1
        d2 = x2 - c2
        d = d0 * d0 + d1 * d1 + d2 * d2
        dist = jnp.minimum(dist_ref[...], d)
        dist_ref[...] = dist
        hit = col == i
        idx_ref[...] = jnp.where(hit, jnp.broadcast_to(far, (_B, _NC)), idx_ref[...])
        nx0_ref[...] = jnp.where(hit, jnp.broadcast_to(c0, (_B, _NC)), nx0_ref[...])
        nx1_ref[...] = jnp.where(hit, jnp.broadcast_to(c1, (_B, _NC)), nx1_ref[...])
        nx2_ref[...] = jnp.where(hit, jnp.broadcast_to(c2, (_B, _NC)), nx2_ref[...])
        far_new = jnp.argmax(dist, axis=1).astype(jnp.int32)[:, None]
        return far_new

    jax.lax.fori_loop(0, _NC, body, far0_ref[...], unroll=2)


def _fps_call(xyz, far0):
    return pl.pallas_call(
        _fps_body,
        out_shape=(
            jax.ShapeDtypeStruct((_B, _NC), jnp.int32),
            jax.ShapeDtypeStruct((_B, _NC), jnp.float32),
            jax.ShapeDtypeStruct((_B, _NC), jnp.float32),
            jax.ShapeDtypeStruct((_B, _NC), jnp.float32),
        ),
        scratch_shapes=[pltpu.VMEM((_B, _N), jnp.float32)],
    )(xyz, far0)


def _lrelu(h):
    return jnp.where(h >= 0, h, 0.2 * h)


def _dot(a, b):
    return jax.lax.dot_general(a, b, (((1,), (0,)), ((), ())),
                               preferred_element_type=jnp.float32)


def _mlp_body(xt_ref, w1_ref, b1_ref, g1_ref, be1_ref, w2_ref, b2_ref, g2_ref,
              be2_ref, w3_ref, b3_ref, w3p_ref, b3p_ref, ft_ref):
    xt = xt_ref[...]
    h = _dot(w1_ref[...], xt) + b1_ref[...]
    m1 = jnp.mean(h, axis=1, keepdims=True)
    v1 = jnp.mean((h - m1) ** 2, axis=1, keepdims=True)
    h = _lrelu((h - m1) / jnp.sqrt(v1 + _EPS) * g1_ref[...] + be1_ref[...])
    h = _dot(w2_ref[...], h) + b2_ref[...]
    m2 = jnp.mean(h, axis=1, keepdims=True)
    v2 = jnp.mean((h - m2) ** 2, axis=1, keepdims=True)
    h = _lrelu((h - m2) / jnp.sqrt(v2 + _EPS) * g2_ref[...] + be2_ref[...])
    ft_ref[...] = jax.lax.dot_general(h, w3p_ref[...], (((0,), (1,)), ((), ())),
                                      preferred_element_type=jnp.float32) \
        + b3p_ref[...]


def _mlp_call(xt, w1, b1, g1, be1, w2, b2, g2, be2, w3, b3, w3p, b3p):
    return pl.pallas_call(
        _mlp_body,
        out_shape=jax.ShapeDtypeStruct((_B * _N, 128), jnp.float32),
    )(xt, w1, b1, g1, be1, w2, b2, g2, be2, w3, b3, w3p, b3p)


_sc_mesh = plsc.VectorSubcoreMesh(core_axis_name="c", subcore_axis_name="s")


@functools.partial(
    pl.kernel,
    out_type=jax.ShapeDtypeStruct((_B * _NC, 128), jnp.float32),
    mesh=_sc_mesh,
    scratch_types=[
        pltpu.VMEM((_ROWS_PER_W,), jnp.int32),
        pltpu.VMEM((_ROWS_PER_W, 128), jnp.float32),
        pltpu.SemaphoreType.DMA,
    ],
)
def _sc_gather(table_hbm, gidx_hbm, out_hbm, idx_v, rows_v, sem):
    wid = lax.axis_index("s") * _SC_CORES + lax.axis_index("c")
    base = wid * _ROWS_PER_W
    pltpu.sync_copy(gidx_hbm.at[pl.ds(base, _ROWS_PER_W)], idx_v)
    pltpu.async_copy(table_hbm.at[idx_v], rows_v, sem).wait()
    pltpu.sync_copy(rows_v, out_hbm.at[pl.ds(base, _ROWS_PER_W)])


def _attn_body(f_ref, cent_ref, wq_ref, wk_ref, wv_ref, wo_ref, out_ref):
    fb = f_ref[:, 0:64]
    cent = cent_ref[0]
    q = jax.lax.dot_general(cent, wq_ref[...], (((1,), (1,)), ((), ())),
                            preferred_element_type=jnp.float32)
    kt = jax.lax.dot_general(fb, wk_ref[...], (((1,), (1,)), ((), ())),
                             preferred_element_type=jnp.float32)
    logits = jax.lax.dot_general(q, kt, (((1,), (1,)), ((), ())),
                                 preferred_element_type=jnp.float32) * 0.125
    mx = jnp.max(logits, axis=1, keepdims=True)
    e = jnp.exp(logits - mx)
    probs = e / jnp.sum(e, axis=1, keepdims=True)
    t = jax.lax.dot_general(probs, fb, (((1,), (0,)), ((), ())),
                            preferred_element_type=jnp.float32)
    o = jax.lax.dot_general(t, wv_ref[...], (((1,), (1,)), ((), ())),
                            preferred_element_type=jnp.float32)
    y = jax.lax.dot_general(o, wo_ref[...], (((1,), (1,)), ((), ())),
                            preferred_element_type=jnp.float32)
    out_ref[0] = cent + y


def _attn_call(f, cent3, wq, wk, wv, wo):
    wspec = pl.BlockSpec((64, 64), lambda b: (0, 0))
    return pl.pallas_call(
        _attn_body,
        grid=(_B,),
        in_specs=[
            pl.BlockSpec((_N, 128), lambda b: (b, 0)),
            pl.BlockSpec((1, _NC, 64), lambda b: (b, 0, 0)),
            wspec, wspec, wspec, wspec,
        ],
        out_specs=pl.BlockSpec((1, _NC, 64), lambda b: (b, 0, 0)),
        out_shape=jax.ShapeDtypeStruct((_B, _NC, 64), jnp.float32),
        compiler_params=pltpu.CompilerParams(
            dimension_semantics=("arbitrary",)),
    )(f, cent3, wq, wk, wv, wo)


def kernel(xyz, W1, b1, g1, be1, W2, b2, g2, be2, W3, b3, Wq, Wk, Wv, Wo):
    far0 = jax.random.randint(jax.random.key(42), (_B,), 0, _N,
                              dtype=jnp.int32).reshape(_B, 1)
    idx, nx0, nx1, nx2 = _fps_call(xyz, far0)
    new_xyz = jnp.stack([nx0, nx1, nx2], axis=1)

    xt = xyz.transpose(1, 0, 2).reshape(3, _B * _N)
    col = lambda a: a.reshape(-1, 1)
    w3p = jnp.zeros((128, 32), jnp.float32).at[:64].set(W3)
    b3p = jnp.zeros((1, 128), jnp.float32).at[0, :64].set(b3)
    ft = _mlp_call(xt, W1, col(b1), col(g1), col(be1), W2, col(b2),
                   col(g2), col(be2), W3, col(b3), w3p, b3p)

    gidx = (jnp.arange(_B, dtype=jnp.int32)[:, None] * _N + idx).reshape(-1)
    cent = _sc_gather(ft, gidx)[:, :64]

    out2t = _attn_call(ft, cent.reshape(_B, _NC, 64), Wq, Wk, Wv, Wo)
    return (new_xyz, out2t.transpose(0, 2, 1))


# speedup vs baseline: 1.2191x; 1.2191x over previous
import jax
import jax.numpy as jnp
from jax.experimental import pallas as pl
from jax.experimental.pallas import tpu as pltpu

_B = 8
_N = 8192
_NC = 512
_EPS = 1e-5


def _fps_body(xyz_ref, far0_ref, idx_ref, nx0_ref, nx1_ref, nx2_ref, dist_ref):
    x0 = xyz_ref[:, 0, :]
    x1 = xyz_ref[:, 1, :]
    x2 = xyz_ref[:, 2, :]
    lane = jax.lax.broadcasted_iota(jnp.int32, (_B, _N), 1)
    col = jax.lax.broadcasted_iota(jnp.int32, (_B, _NC), 1)
    dist_ref[...] = jnp.full((_B, _N), 1e10, jnp.float32)
    idx_ref[...] = jnp.zeros((_B, _NC), jnp.int32)
    nx0_ref[...] = jnp.zeros((_B, _NC), jnp.float32)
    nx1_ref[...] = jnp.zeros((_B, _NC), jnp.float32)
    nx2_ref[...] = jnp.zeros((_B, _NC), jnp.float32)

    x24 = jnp.concatenate([x0, x1, x2], axis=0)

    def body(i, far):
        sel = lane == far
        sel24 = jnp.concatenate([sel, sel, sel], axis=0)
        g = jnp.sum(jnp.where(sel24, x24, 0.0), axis=1, keepdims=True)
        c0 = g[0:_B]
        c1 = g[_B:2 * _B]
        c2 = g[2 * _B:3 * _B]
        d0 = x0 - c0
        d1 = x1 - c1
        d2 = x2 - c2
        d = d0 * d0 + d1 * d1 + d2 * d2
        dist = jnp.minimum(dist_ref[...], d)
        dist_ref[...] = dist
        hit = col == i
        idx_ref[...] = jnp.where(hit, jnp.broadcast_to(far, (_B, _NC)), idx_ref[...])
        nx0_ref[...] = jnp.where(hit, jnp.broadcast_to(c0, (_B, _NC)), nx0_ref[...])
        nx1_ref[...] = jnp.where(hit, jnp.broadcast_to(c1, (_B, _NC)), nx1_ref[...])
        nx2_ref[...] = jnp.where(hit, jnp.broadcast_to(c2, (_B, _NC)), nx2_ref[...])
        far_new = jnp.argmax(dist, axis=1).astype(jnp.int32)[:, None]
        return far_new

    jax.lax.fori_loop(0, _NC, body, far0_ref[...], unroll=4)


def _fps_call(xyz, far0):
    return pl.pallas_call(
        _fps_body,
        out_shape=(
            jax.ShapeDtypeStruct((_B, _NC), jnp.int32),
            jax.ShapeDtypeStruct((_B, _NC), jnp.float32),
            jax.ShapeDtypeStruct((_B, _NC), jnp.float32),
            jax.ShapeDtypeStruct((_B, _NC), jnp.float32),
        ),
        scratch_shapes=[pltpu.VMEM((_B, _N), jnp.float32)],
    )(xyz, far0)


def _lrelu(h):
    return jnp.where(h >= 0, h, 0.2 * h)


def _dot(a, b):
    return jax.lax.dot_general(a, b, (((1,), (0,)), ((), ())),
                               preferred_element_type=jnp.float32)


def _net_body(xt_ref, xc_ref, w1_ref, b1_ref, g1_ref, be1_ref, w2_ref, b2_ref,
              g2_ref, be2_ref, w3_ref, b3_ref, wq_ref, wk_ref, wv_ref, wo_ref,
              out_ref, f_scr, st_scr):
    b = pl.program_id(0)

    @pl.when(b == 0)
    def _mlp():
        xt = xt_ref[...]
        h = _dot(w1_ref[...], xt) + b1_ref[...]
        m1 = jnp.mean(h, axis=1, keepdims=True)
        v1 = jnp.mean((h - m1) ** 2, axis=1, keepdims=True)
        h = _lrelu((h - m1) / jnp.sqrt(v1 + _EPS) * g1_ref[...] + be1_ref[...])
        h = _dot(w2_ref[...], h) + b2_ref[...]
        m2 = jnp.mean(h, axis=1, keepdims=True)
        v2 = jnp.mean((h - m2) ** 2, axis=1, keepdims=True)
        h = _lrelu((h - m2) / jnp.sqrt(v2 + _EPS) * g2_ref[...] + be2_ref[...])
        f_scr[...] = _dot(w3_ref[...], h) + b3_ref[...]
        st_scr[:, 0:1] = m1
        st_scr[:, 1:2] = v1
        st_scr[:, 2:3] = m2
        st_scr[:, 3:4] = v2

    fb = f_scr[:, pl.ds(b * _N, _N)]

    m1 = st_scr[:, 0:1]
    v1 = st_scr[:, 1:2]
    m2 = st_scr[:, 2:3]
    v2 = st_scr[:, 3:4]
    hc = _dot(w1_ref[...], xc_ref[0]) + b1_ref[...]
    hc = _lrelu((hc - m1) / jnp.sqrt(v1 + _EPS) * g1_ref[...] + be1_ref[...])
    hc = _dot(w2_ref[...], hc) + b2_ref[...]
    hc = _lrelu((hc - m2) / jnp.sqrt(v2 + _EPS) * g2_ref[...] + be2_ref[...])
    cent = _dot(w3_ref[...], hc) + b3_ref[...]

    q = _dot(wq_ref[...], cent)
    k = _dot(wk_ref[...], fb)
    v = _dot(wv_ref[...], fb)
    logits_t = jax.lax.dot_general(k, q, (((0,), (0,)), ((), ())),
                                   preferred_element_type=jnp.float32) * 0.125
    mx = jnp.max(logits_t, axis=0, keepdims=True)
    e = jnp.exp(logits_t - mx)
    probs_t = e / jnp.sum(e, axis=0, keepdims=True)
    o = jax.lax.dot_general(v, probs_t, (((1,), (0,)), ((), ())),
                            preferred_element_type=jnp.float32)
    y = _dot(wo_ref[...], o)
    out_ref[0] = cent + y


def _net_call(xt, nxc, w1, b1, g1, be1, w2, b2, g2, be2, w3, b3, wq, wk, wv, wo):
    full = lambda a: pl.BlockSpec(a.shape, lambda b: (0,) * a.ndim)
    return pl.pallas_call(
        _net_body,
        grid=(_B,),
        in_specs=[
            full(xt),
            pl.BlockSpec((1, 3, _NC), lambda b: (b, 0, 0)),
        ] + [full(w) for w in (w1, b1, g1, be1, w2, b2, g2, be2, w3, b3,
                               wq, wk, wv, wo)],
        out_specs=pl.BlockSpec((1, 64, _NC), lambda b: (b, 0, 0)),
        out_shape=jax.ShapeDtypeStruct((_B, 64, _NC), jnp.float32),
        scratch_shapes=[pltpu.VMEM((64, _B * _N), jnp.float32),
                        pltpu.VMEM((32, 4), jnp.float32)],
        compiler_params=pltpu.CompilerParams(
            dimension_semantics=("arbitrary",)),
    )(xt, nxc, w1, b1, g1, be1, w2, b2, g2, be2, w3, b3, wq, wk, wv, wo)


def kernel(xyz, W1, b1, g1, be1, W2, b2, g2, be2, W3, b3, Wq, Wk, Wv, Wo):
    far0 = jax.random.randint(jax.random.key(42), (_B,), 0, _N,
                              dtype=jnp.int32).reshape(_B, 1)
    idx, nx0, nx1, nx2 = _fps_call(xyz, far0)
    new_xyz = jnp.stack([nx0, nx1, nx2], axis=1)

    xt = xyz.transpose(1, 0, 2).reshape(3, _B * _N)
    col = lambda a: a.reshape(-1, 1)
    out2 = _net_call(xt, new_xyz, W1, col(b1), col(g1), col(be1), W2, col(b2),
                     col(g2), col(be2), W3, col(b3), Wq, Wk, Wv, Wo)
    return (new_xyz, out2)


# speedup vs baseline: 1.2246x; 1.0045x over previous
import jax
import jax.numpy as jnp
from jax.experimental import pallas as pl
from jax.experimental.pallas import tpu as pltpu

_B = 8
_N = 8192
_NC = 512
_EPS = 1e-5


def _fps_body(xyz_ref, far0_ref, idx_ref, nx0_ref, nx1_ref, nx2_ref, dist_ref):
    x0 = xyz_ref[:, 0, :]
    x1 = xyz_ref[:, 1, :]
    x2 = xyz_ref[:, 2, :]
    lane = jax.lax.broadcasted_iota(jnp.int32, (_B, _N), 1)
    col = jax.lax.broadcasted_iota(jnp.int32, (_B, _NC), 1)
    dist_ref[...] = jnp.full((_B, _N), 1e10, jnp.float32)
    idx_ref[...] = jnp.zeros((_B, _NC), jnp.int32)
    nx0_ref[...] = jnp.zeros((_B, _NC), jnp.float32)
    nx1_ref[...] = jnp.zeros((_B, _NC), jnp.float32)
    nx2_ref[...] = jnp.zeros((_B, _NC), jnp.float32)

    x24 = jnp.concatenate([x0, x1, x2], axis=0)

    def body(i, far):
        sel = lane == far
        sel24 = jnp.concatenate([sel, sel, sel], axis=0)
        g = jnp.sum(jnp.where(sel24, x24, 0.0), axis=1, keepdims=True)
        c0 = g[0:_B]
        c1 = g[_B:2 * _B]
        c2 = g[2 * _B:3 * _B]
        d0 = x0 - c0
        d1 = x1 - c1
        d2 = x2 - c2
        d = d0 * d0 + d1 * d1 + d2 * d2
        dist = jnp.minimum(dist_ref[...], d)
        dist_ref[...] = dist
        hit = col == i
        idx_ref[...] = jnp.where(hit, jnp.broadcast_to(far, (_B, _NC)), idx_ref[...])
        nx0_ref[...] = jnp.where(hit, jnp.broadcast_to(c0, (_B, _NC)), nx0_ref[...])
        nx1_ref[...] = jnp.where(hit, jnp.broadcast_to(c1, (_B, _NC)), nx1_ref[...])
        nx2_ref[...] = jnp.where(hit, jnp.broadcast_to(c2, (_B, _NC)), nx2_ref[...])
        far_new = jnp.argmax(dist, axis=1).astype(jnp.int32)[:, None]
        return far_new

    jax.lax.fori_loop(0, _NC, body, far0_ref[...], unroll=8)


def _fps_call(xyz, far0):
    return pl.pallas_call(
        _fps_body,
        out_shape=(
            jax.ShapeDtypeStruct((_B, _NC), jnp.int32),
            jax.ShapeDtypeStruct((_B, _NC), jnp.float32),
            jax.ShapeDtypeStruct((_B, _NC), jnp.float32),
            jax.ShapeDtypeStruct((_B, _NC), jnp.float32),
        ),
        scratch_shapes=[pltpu.VMEM((_B, _N), jnp.float32)],
    )(xyz, far0)


def _lrelu(h):
    return jnp.where(h >= 0, h, 0.2 * h)


def _dot(a, b):
    return jax.lax.dot_general(a, b, (((1,), (0,)), ((), ())),
                               preferred_element_type=jnp.float32)


def _net_body(xt_ref, xc_ref, w1_ref, b1_ref, g1_ref, be1_ref, w2_ref, b2_ref,
              g2_ref, be2_ref, w3_ref, b3_ref, wq_ref, wk_ref, wv_ref, wo_ref,
              out_ref, f_scr, st_scr):
    b = pl.program_id(0)

    @pl.when(b == 0)
    def _mlp():
        xt = xt_ref[...]
        h = _dot(w1_ref[...], xt) + b1_ref[...]
        m1 = jnp.mean(h, axis=1, keepdims=True)
        v1 = jnp.mean((h - m1) ** 2, axis=1, keepdims=True)
        h = _lrelu((h - m1) / jnp.sqrt(v1 + _EPS) * g1_ref[...] + be1_ref[...])
        h = _dot(w2_ref[...], h) + b2_ref[...]
        m2 = jnp.mean(h, axis=1, keepdims=True)
        v2 = jnp.mean((h - m2) ** 2, axis=1, keepdims=True)
        h = _lrelu((h - m2) / jnp.sqrt(v2 + _EPS) * g2_ref[...] + be2_ref[...])
        f_scr[...] = _dot(w3_ref[...], h) + b3_ref[...]
        st_scr[:, 0:1] = m1
        st_scr[:, 1:2] = v1
        st_scr[:, 2:3] = m2
        st_scr[:, 3:4] = v2

    fb = f_scr[:, pl.ds(b * _N, _N)]

    m1 = st_scr[:, 0:1]
    v1 = st_scr[:, 1:2]
    m2 = st_scr[:, 2:3]
    v2 = st_scr[:, 3:4]
    hc = _dot(w1_ref[...], xc_ref[0]) + b1_ref[...]
    hc = _lrelu((hc - m1) / jnp.sqrt(v1 + _EPS) * g1_ref[...] + be1_ref[...])
    hc = _dot(w2_ref[...], hc) + b2_ref[...]
    hc = _lrelu((hc - m2) / jnp.sqrt(v2 + _EPS) * g2_ref[...] + be2_ref[...])
    cent = _dot(w3_ref[...], hc) + b3_ref[...]

    q = _dot(wq_ref[...], cent)
    k = _dot(wk_ref[...], fb)
    v = _dot(wv_ref[...], fb)
    logits_t = jax.lax.dot_general(k, q, (((0,), (0,)), ((), ())),
                                   preferred_element_type=jnp.float32) * 0.125
    mx = jnp.max(logits_t, axis=0, keepdims=True)
    e = jnp.exp(logits_t - mx)
    probs_t = e / jnp.sum(e, axis=0, keepdims=True)
    o = jax.lax.dot_general(v, probs_t, (((1,), (0,)), ((), ())),
                            preferred_element_type=jnp.float32)
    y = _dot(wo_ref[...], o)
    out_ref[0] = cent + y


def _net_call(xt, nxc, w1, b1, g1, be1, w2, b2, g2, be2, w3, b3, wq, wk, wv, wo):
    full = lambda a: pl.BlockSpec(a.shape, lambda b: (0,) * a.ndim)
    return pl.pallas_call(
        _net_body,
        grid=(_B,),
        in_specs=[
            full(xt),
            pl.BlockSpec((1, 3, _NC), lambda b: (b, 0, 0)),
        ] + [full(w) for w in (w1, b1, g1, be1, w2, b2, g2, be2, w3, b3,
                               wq, wk, wv, wo)],
        out_specs=pl.BlockSpec((1, 64, _NC), lambda b: (b, 0, 0)),
        out_shape=jax.ShapeDtypeStruct((_B, 64, _NC), jnp.float32),
        scratch_shapes=[pltpu.VMEM((64, _B * _N), jnp.float32),
                        pltpu.VMEM((32, 4), jnp.float32)],
        compiler_params=pltpu.CompilerParams(
            dimension_semantics=("arbitrary",)),
    )(xt, nxc, w1, b1, g1, be1, w2, b2, g2, be2, w3, b3, wq, wk, wv, wo)


def kernel(xyz, W1, b1, g1, be1, W2, b2, g2, be2, W3, b3, Wq, Wk, Wv, Wo):
    far0 = jax.random.randint(jax.random.key(42), (_B,), 0, _N,
                              dtype=jnp.int32).reshape(_B, 1)
    idx, nx0, nx1, nx2 = _fps_call(xyz, far0)
    new_xyz = jnp.stack([nx0, nx1, nx2], axis=1)

    xt = xyz.transpose(1, 0, 2).reshape(3, _B * _N)
    col = lambda a: a.reshape(-1, 1)
    out2 = _net_call(xt, new_xyz, W1, col(b1), col(g1), col(be1), W2, col(b2),
                     col(g2), col(be2), W3, col(b3), Wq, Wk, Wv, Wo)
    return (new_xyz, out2)


# speedup vs baseline: 1.2946x; 1.0572x over previous
import jax
import jax.numpy as jnp
from jax.experimental import pallas as pl
from jax.experimental.pallas import tpu as pltpu

_B = 8
_N = 8192
_NC = 512
_EPS = 1e-5


_FAR0 = (5316, 4114, 1207, 7361, 653, 7531, 2433, 2343)


def _fps_body(xyz_ref, packed_ref, dist_ref):
    x0 = xyz_ref[:, 0, :]
    x1 = xyz_ref[:, 1, :]
    x2 = xyz_ref[:, 2, :]
    lane = jax.lax.broadcasted_iota(jnp.int32, (_B, _N), 1)
    col32 = jax.lax.broadcasted_iota(jnp.int32, (4 * _B, _NC), 1)
    dist_ref[...] = jnp.full((_B, _N), 1e10, jnp.float32)
    packed_ref[...] = jnp.zeros((4 * _B, _NC), jnp.float32)

    x24 = jnp.concatenate([x0, x1, x2], axis=0)
    row = jax.lax.broadcasted_iota(jnp.int32, (_B, 1), 0)
    far0 = jnp.zeros((_B, 1), jnp.int32)
    for b, v in enumerate(_FAR0):
        far0 = jnp.where(row == b, v, far0)

    def body(i, far):
        sel = lane == far
        sel24 = jnp.concatenate([sel, sel, sel], axis=0)
        g = jnp.sum(jnp.where(sel24, x24, 0.0), axis=1, keepdims=True)
        c0 = g[0:_B]
        c1 = g[_B:2 * _B]
        c2 = g[2 * _B:3 * _B]
        d0 = x0 - c0
        d1 = x1 - c1
        d2 = x2 - c2
        d = d0 * d0 + d1 * d1 + d2 * d2
        dist = jnp.minimum(dist_ref[...], d)
        dist_ref[...] = dist
        far_f = jax.lax.bitcast_convert_type(far, jnp.float32)
        val = jnp.concatenate([far_f, g], axis=0)
        packed_ref[...] = jnp.where(col32 == i,
                                    jnp.broadcast_to(val, (4 * _B, _NC)),
                                    packed_ref[...])
        far_new = jnp.argmax(dist, axis=1).astype(jnp.int32)[:, None]
        return far_new

    jax.lax.fori_loop(0, _NC, body, far0, unroll=8)


def _fps_call(xyz):
    return pl.pallas_call(
        _fps_body,
        out_shape=jax.ShapeDtypeStruct((4 * _B, _NC), jnp.float32),
        scratch_shapes=[pltpu.VMEM((_B, _N), jnp.float32)],
    )(xyz)


def _lrelu(h):
    return jnp.where(h >= 0, h, 0.2 * h)


def _dot(a, b):
    return jax.lax.dot_general(a, b, (((1,), (0,)), ((), ())),
                               preferred_element_type=jnp.float32)


def _net_body(xt_ref, xc_ref, w1_ref, b1_ref, g1_ref, be1_ref, w2_ref, b2_ref,
              g2_ref, be2_ref, w3_ref, b3_ref, wq_ref, wk_ref, wv_ref, wo_ref,
              out_ref, f_scr, st_scr):
    b = pl.program_id(0)

    @pl.when(b == 0)
    def _mlp():
        xt = xt_ref[...]
        h = _dot(w1_ref[...], xt) + b1_ref[...]
        m1 = jnp.mean(h, axis=1, keepdims=True)
        v1 = jnp.mean((h - m1) ** 2, axis=1, keepdims=True)
        h = _lrelu((h - m1) / jnp.sqrt(v1 + _EPS) * g1_ref[...] + be1_ref[...])
        h = _dot(w2_ref[...], h) + b2_ref[...]
        m2 = jnp.mean(h, axis=1, keepdims=True)
        v2 = jnp.mean((h - m2) ** 2, axis=1, keepdims=True)
        h = _lrelu((h - m2) / jnp.sqrt(v2 + _EPS) * g2_ref[...] + be2_ref[...])
        f_scr[...] = _dot(w3_ref[...], h) + b3_ref[...]
        st_scr[:, 0:1] = m1
        st_scr[:, 1:2] = v1
        st_scr[:, 2:3] = m2
        st_scr[:, 3:4] = v2

    fb = f_scr[:, pl.ds(b * _N, _N)]

    m1 = st_scr[:, 0:1]
    v1 = st_scr[:, 1:2]
    m2 = st_scr[:, 2:3]
    v2 = st_scr[:, 3:4]
    hc = _dot(w1_ref[...], xc_ref[0]) + b1_ref[...]
    hc = _lrelu((hc - m1) / jnp.sqrt(v1 + _EPS) * g1_ref[...] + be1_ref[...])
    hc = _dot(w2_ref[...], hc) + b2_ref[...]
    hc = _lrelu((hc - m2) / jnp.sqrt(v2 + _EPS) * g2_ref[...] + be2_ref[...])
    cent = _dot(w3_ref[...], hc) + b3_ref[...]

    q = _dot(wq_ref[...], cent)
    k = _dot(wk_ref[...], fb)
    v = _dot(wv_ref[...], fb)
    logits_t = jax.lax.dot_general(k, q, (((0,), (0,)), ((), ())),
                                   preferred_element_type=jnp.float32) * 0.125
    mx = jnp.max(logits_t, axis=0, keepdims=True)
    e = jnp.exp(logits_t - mx)
    probs_t = e / jnp.sum(e, axis=0, keepdims=True)
    o = jax.lax.dot_general(v, probs_t, (((1,), (0,)), ((), ())),
                            preferred_element_type=jnp.float32)
    y = _dot(wo_ref[...], o)
    out_ref[0] = cent + y


def _net_call(xt, nxc, w1, b1, g1, be1, w2, b2, g2, be2, w3, b3, wq, wk, wv, wo):
    full = lambda a: pl.BlockSpec(a.shape, lambda b: (0,) * a.ndim)
    return pl.pallas_call(
        _net_body,
        grid=(_B,),
        in_specs=[
            full(xt),
            pl.BlockSpec((1, 3, _NC), lambda b: (b, 0, 0)),
        ] + [full(w) for w in (w1, b1, g1, be1, w2, b2, g2, be2, w3, b3,
                               wq, wk, wv, wo)],
        out_specs=pl.BlockSpec((1, 64, _NC), lambda b: (b, 0, 0)),
        out_shape=jax.ShapeDtypeStruct((_B, 64, _NC), jnp.float32),
        scratch_shapes=[pltpu.VMEM((64, _B * _N), jnp.float32),
                        pltpu.VMEM((32, 4), jnp.float32)],
        compiler_params=pltpu.CompilerParams(
            dimension_semantics=("arbitrary",)),
    )(xt, nxc, w1, b1, g1, be1, w2, b2, g2, be2, w3, b3, wq, wk, wv, wo)


def kernel(xyz, W1, b1, g1, be1, W2, b2, g2, be2, W3, b3, Wq, Wk, Wv, Wo):
    packed = _fps_call(xyz)
    idx = jax.lax.bitcast_convert_type(packed[0:_B], jnp.int32)
    new_xyz = packed[_B:4 * _B].reshape(3, _B, _NC).transpose(1, 0, 2)

    xt = xyz.transpose(1, 0, 2).reshape(3, _B * _N)
    col = lambda a: a.reshape(-1, 1)
    out2 = _net_call(xt, new_xyz, W1, col(b1), col(g1), col(be1), W2, col(b2),
                     col(g2), col(be2), W3, col(b3), Wq, Wk, Wv, Wo)
    return (new_xyz, out2)
